# baseline (speedup 1.0000x reference)
import functools

import jax
import jax.numpy as jnp
from jax import lax
from jax.experimental import pallas as pl
from jax.experimental.pallas import tpu as pltpu
from jax.experimental.pallas import tpu_sc as plsc

_B = 4096
_D = 512
_K = 1000
_LANES = 16
_NC = 2
_NS = 16
_NW = _NC * _NS
_BPW = _B // _NW
_RPT = 64
_HALF = _RPT // 2
_SCALE = 0.01 / _B

_mesh = plsc.VectorSubcoreMesh(core_axis_name="c", subcore_axis_name="s")


@functools.partial(
    pl.kernel,
    out_type=jax.ShapeDtypeStruct((_NW, _LANES), jnp.float32),
    mesh=_mesh,
    scratch_types=[
        pltpu.VMEM((2, _HALF, _D), jnp.float32),
        pltpu.VMEM((_RPT * _LANES,), jnp.float32),
        pltpu.VMEM((_K * _LANES,), jnp.float32),
        pltpu.VMEM((_BPW,), jnp.int32),
        pltpu.VMEM((_LANES,), jnp.float32),
        pltpu.VMEM_SHARED((_K * _LANES,), jnp.float32),
        pltpu.SemaphoreType.DMA,
        pltpu.SemaphoreType.DMA,
    ],
)
def _norm_gather_sc(lab_hbm, cent_hbm, out_hbm,
                    crows_v, p_v, tbl_v, lab_v, acc_v, tbl_sh, sem_l, sem_c):
    c = lax.axis_index("c")
    s = lax.axis_index("s")
    wid = s * _NC + c
    row_start = jnp.where(s == _NS - 1, _K - _RPT, s * _RPT)

    lab_cp = pltpu.async_copy(lab_hbm.at[pl.ds(wid * _BPW, _BPW)], lab_v, sem_l)
    cp0 = pltpu.async_copy(
        cent_hbm.at[pl.ds(row_start, _HALF)], crows_v.at[0], sem_c)
    cp1 = pltpu.async_copy(
        cent_hbm.at[pl.ds(row_start + _HALF, _HALF)], crows_v.at[1], sem_c)

    for half, cp in ((0, cp0), (1, cp1)):
        cp.wait()

        def quad_body(i, carry, half=half):
            for u in range(4):
                r = i * 4 + u
                acc = jnp.zeros((_LANES,), jnp.float32)
                for k in range(_D // _LANES):
                    v = crows_v[half, r, pl.ds(k * _LANES, _LANES)]
                    acc = acc + v * v
                p_v[pl.ds((half * _HALF + r) * _LANES, _LANES)] = acc
            return carry

        lax.fori_loop(0, _HALF // 4, quad_body, 0)

    pltpu.sync_copy(p_v, tbl_sh.at[pl.ds(row_start * _LANES, _RPT * _LANES)])
    plsc.subcore_barrier()
    pltpu.sync_copy(tbl_sh, tbl_v)

    lab_cp.wait()

    def lab_body(i, tot):
        lv = lab_v[pl.ds(i * _LANES, _LANES)] * _LANES
        for u in range(_LANES):
            tot = tot + tbl_v[pl.ds(lv[u], _LANES)]
        return tot

    tot = lax.fori_loop(0, _BPW // _LANES, lab_body,
                        jnp.zeros((_LANES,), jnp.float32))
    acc_v[...] = tot
    pltpu.sync_copy(acc_v, out_hbm.at[wid])


_BLK = 1024
_GRID = _B // _BLK


def _cross_tc_body(lab_ref, feat_ref, centb_ref, sc_ref, out_ref):
    i = pl.program_id(0)
    f = feat_ref[...]
    lab = lab_ref[0, 0, :]
    oh = (lab[:, None] == lax.broadcasted_iota(jnp.int32, (_BLK, _K), 1))
    cb = lax.dot_general(
        oh.astype(jnp.bfloat16), centb_ref[...],
        (((1,), (0,)), ((), ())), preferred_element_type=jnp.float32)
    s = jnp.sum(f * (f - 2.0 * cb))

    @pl.when(i == 0)
    def _init():
        out_ref[0, 0] = s + jnp.sum(sc_ref[...])

    @pl.when(i > 0)
    def _acc():
        out_ref[0, 0] = out_ref[0, 0] + s


_cross_tc = pl.pallas_call(
    _cross_tc_body,
    grid=(_GRID,),
    in_specs=[
        pl.BlockSpec((1, 1, _BLK), lambda i: (i, 0, 0)),
        pl.BlockSpec((_BLK, _D), lambda i: (i, 0)),
        pl.BlockSpec((_K, _D), lambda i: (0, 0)),
        pl.BlockSpec((_NW, _LANES), lambda i: (0, 0)),
    ],
    out_specs=pl.BlockSpec((1, 1), lambda i: (0, 0),
                           memory_space=pltpu.SMEM),
    out_shape=jax.ShapeDtypeStruct((1, 1), jnp.float32),
)


def kernel(features, labels, centers):
    labels = labels.astype(jnp.int32)
    sc_part = _norm_gather_sc(labels, centers)
    total = _cross_tc(labels.reshape(_GRID, 1, _BLK), features,
                      centers.astype(jnp.bfloat16), sc_part)
    return _SCALE * total[0, 0]


# speedup vs baseline: 1.2311x; 1.2311x over previous
import functools

import jax
import jax.numpy as jnp
from jax import lax
from jax.experimental import pallas as pl
from jax.experimental.pallas import tpu as pltpu
from jax.experimental.pallas import tpu_sc as plsc

_B = 4096
_D = 512
_K = 1000
_LANES = 16
_NC = 2
_NS = 16
_NW = _NC * _NS
_BPW = _B // _NW
_RPT = 64
_HALF = _RPT // 2
_SCALE = 0.01 / _B

_mesh = plsc.VectorSubcoreMesh(core_axis_name="c", subcore_axis_name="s")


@functools.partial(
    pl.kernel,
    out_type=jax.ShapeDtypeStruct((_NW, _LANES), jnp.float32),
    mesh=_mesh,
    scratch_types=[
        pltpu.VMEM((2, _HALF, _D), jnp.float32),
        pltpu.VMEM((_RPT * _LANES,), jnp.float32),
        pltpu.VMEM((_K * _LANES,), jnp.float32),
        pltpu.VMEM((_BPW,), jnp.int32),
        pltpu.VMEM((_LANES,), jnp.float32),
        pltpu.VMEM_SHARED((_K * _LANES,), jnp.float32),
        pltpu.SemaphoreType.DMA,
        pltpu.SemaphoreType.DMA,
    ],
)
def _norm_gather_sc(lab_hbm, cent_hbm, out_hbm,
                    crows_v, p_v, tbl_v, lab_v, acc_v, tbl_sh, sem_l, sem_c):
    c = lax.axis_index("c")
    s = lax.axis_index("s")
    wid = s * _NC + c
    row_start = jnp.where(s == _NS - 1, _K - _RPT, s * _RPT)

    lab_cp = pltpu.async_copy(lab_hbm.at[pl.ds(wid * _BPW, _BPW)], lab_v, sem_l)
    cp0 = pltpu.async_copy(
        cent_hbm.at[pl.ds(row_start, _HALF)], crows_v.at[0], sem_c)
    cp1 = pltpu.async_copy(
        cent_hbm.at[pl.ds(row_start + _HALF, _HALF)], crows_v.at[1], sem_c)

    for half, cp in ((0, cp0), (1, cp1)):
        cp.wait()

        def quad_body(i, carry, half=half):
            for u in range(4):
                r = i * 4 + u
                acc = jnp.zeros((_LANES,), jnp.float32)
                for k in range(_D // _LANES):
                    v = crows_v[half, r, pl.ds(k * _LANES, _LANES)]
                    acc = acc + v * v
                p_v[pl.ds((half * _HALF + r) * _LANES, _LANES)] = acc
            return carry

        lax.fori_loop(0, _HALF // 4, quad_body, 0)

    pltpu.sync_copy(p_v, tbl_sh.at[pl.ds(row_start * _LANES, _RPT * _LANES)])
    plsc.subcore_barrier()
    pltpu.sync_copy(tbl_sh, tbl_v)

    lab_cp.wait()

    def lab_body(i, tot):
        lv = lab_v[pl.ds(i * _LANES, _LANES)] * _LANES
        for u in range(_LANES):
            tot = tot + tbl_v[pl.ds(lv[u], _LANES)]
        return tot

    tot = lax.fori_loop(0, _BPW // _LANES, lab_body,
                        jnp.zeros((_LANES,), jnp.float32))
    acc_v[...] = tot
    pltpu.sync_copy(acc_v, out_hbm.at[wid])


_BLK = 1024
_GRID = _B // _BLK


def _cross_tc_body(lab_ref, feat_ref, cent_ref, out_ref):
    f = feat_ref[...]
    lab = lab_ref[0, 0, :]
    oh = (lab[:, None] == lax.broadcasted_iota(jnp.int32, (_BLK, _K), 1))
    cb = lax.dot_general(
        oh.astype(jnp.bfloat16), cent_ref[...].astype(jnp.bfloat16),
        (((1,), (0,)), ((), ())), preferred_element_type=jnp.float32)
    t = f * (f - 2.0 * cb)
    out_ref[...] = jnp.sum(t, axis=0).reshape(1, 1, _D)


_cross_tc = pl.pallas_call(
    _cross_tc_body,
    grid=(_GRID,),
    in_specs=[
        pl.BlockSpec((1, 1, _BLK), lambda i: (i, 0, 0)),
        pl.BlockSpec((_BLK, _D), lambda i: (i, 0)),
        pl.BlockSpec((_K, _D), lambda i: (0, 0)),
    ],
    out_specs=pl.BlockSpec((1, 1, _D), lambda i: (i, 0, 0)),
    out_shape=jax.ShapeDtypeStruct((_GRID, 1, _D), jnp.float32),
)


def kernel(features, labels, centers):
    labels = labels.astype(jnp.int32)
    sc_part = _norm_gather_sc(labels, centers)
    tc_part = _cross_tc(labels.reshape(_GRID, 1, _BLK), features, centers)
    return _SCALE * (jnp.sum(sc_part) + jnp.sum(tc_part))
